# baseline (speedup 1.0000x reference)
import jax
import jax.numpy as jnp
from jax import lax
from jax.experimental import pallas as pl
from jax.experimental.pallas import tpu as pltpu
from jax.experimental.pallas import tpu_sc as plsc

N_NODES = 10000
N_EDGES = 320000
D = 128
N_T = 17
NT = N_NODES * N_T
NC, NS, L = 2, 16, 16
NW = NC * NS
EPW = N_EDGES // NW
B = 80
NCHUNK = EPW // B
ZH = 10752
NT_PAD = ZH * NS
N_PAD = 10240
RPT = N_PAD // NS
ZROW = 128

_mesh = plsc.VectorSubcoreMesh(core_axis_name="c", subcore_axis_name="s")


def _z16f():
    return jnp.zeros((L,), jnp.float32)


def _bcast_lane(vec, lane):
    idx = jnp.full((L, 1), lane, jnp.int32)
    return lax.gather(
        vec, idx,
        lax.GatherDimensionNumbers(offset_dims=(), collapsed_slice_dims=(0,),
                                   start_index_map=(0,)),
        (1,), mode=lax.GatherScatterMode.PROMISE_IN_BOUNDS)


def _hist_body(ei2_hbm, et_hbm, out0, out1,
               ei_v0, ei_v1, seg_v0, seg_v1, ones_v, zbuf, hist_sh,
               semei0, semei1, sems0, sems1):
    cid = lax.axis_index("c")
    sid = lax.axis_index("s")
    wid = cid * NS + sid
    ei = (ei_v0, ei_v1)
    seg = (seg_v0, seg_v1)
    semei = (semei0, semei1)
    sems = (sems0, sems1)

    def zb(i, carry):
        zbuf[pl.ds(i * L, L)] = _z16f()
        return carry

    lax.fori_loop(0, ZH // L, zb, 0)
    pltpu.sync_copy(zbuf, hist_sh.at[pl.ds(sid * ZH, ZH)])
    for j in range(B // L):
        ones_v[pl.ds(j * L, L)] = jnp.ones((L,), jnp.float32)
    plsc.subcore_barrier()

    def load_ei(k, p):
        base = wid * EPW + k * B
        pltpu.async_copy(ei2_hbm.at[pl.ds(N_EDGES + base, B)], ei[p].at[1], semei[p])
        pltpu.async_copy(et_hbm.at[pl.ds(base, B)], ei[p].at[2], semei[p])

    def wait_ei(p):
        pltpu.make_async_copy(ei2_hbm.at[pl.ds(0, B)], ei[p].at[1],
                              semei[p]).wait()
        pltpu.make_async_copy(et_hbm.at[pl.ds(0, B)], ei[p].at[2],
                              semei[p]).wait()

    load_ei(0, 0)

    def step(k, p):
        @pl.when(k < NCHUNK)
        def _():
            wait_ei(p)

            @pl.when(k >= 2)
            def _():
                pltpu.make_async_copy(ones_v, hist_sh.at[seg[p]],
                                      sems[p]).wait()

            for j in range(B // L):
                s = pl.ds(j * L, L)
                seg[p][s] = ei[p][1, s] * N_T + ei[p][2, s]

            @pl.when(k + 1 < NCHUNK)
            def _():
                load_ei(jnp.minimum(k + 1, NCHUNK - 1), 1 - p)

            pltpu.async_copy(ones_v, hist_sh.at[seg[p]], sems[p], add=True)

    def body(k2, carry):
        for b in range(2):
            step(2 * k2 + b, b)
        return carry

    lax.fori_loop(0, (NCHUNK + 1) // 2, body, 0)
    pltpu.make_async_copy(ones_v, hist_sh.at[seg[0]], sems[0]).wait()
    pltpu.make_async_copy(ones_v, hist_sh.at[seg[1]], sems[1]).wait()
    plsc.subcore_barrier()
    pltpu.sync_copy(hist_sh.at[pl.ds(sid * ZH, ZH)], zbuf)

    @pl.when(cid == 0)
    def _():
        pltpu.sync_copy(zbuf, out0.at[pl.ds(sid * ZH, ZH)])

    @pl.when(cid == 1)
    def _():
        pltpu.sync_copy(zbuf, out1.at[pl.ds(sid * ZH, ZH)])


_hist = pl.kernel(
    _hist_body,
    out_type=(jax.ShapeDtypeStruct((NT_PAD,), jnp.float32),
              jax.ShapeDtypeStruct((NT_PAD,), jnp.float32)),
    mesh=_mesh,
    scratch_types=[
        pltpu.VMEM((3, B), jnp.int32),
        pltpu.VMEM((3, B), jnp.int32),
        pltpu.VMEM((B,), jnp.int32),
        pltpu.VMEM((B,), jnp.int32),
        pltpu.VMEM((B,), jnp.float32),
        pltpu.VMEM((ZH,), jnp.float32),
        pltpu.VMEM_SHARED((NT_PAD,), jnp.float32),
        pltpu.SemaphoreType.DMA,
        pltpu.SemaphoreType.DMA,
        pltpu.SemaphoreType.DMA,
        pltpu.SemaphoreType.DMA,
    ],
)


def _scat_body(ei2_hbm, et_hbm, xat_hbm, rec_hbm, outp0, outp1,
               ei_v0, ei_v1, gid0, gid1, sg0, sg1, dc0, dc1, r_0, r_1,
               rw0, rw1, zbuf, h_sh,
               semei0, semei1, semg0, semg1, sems0, sems1):
    cid = lax.axis_index("c")
    sid = lax.axis_index("s")
    wid = cid * NS + sid
    ei = (ei_v0, ei_v1)
    gid = (gid0, gid1)
    sg = (sg0, sg1)
    dc = (dc0, dc1)
    rr = (r_0, r_1)
    rw = (rw0, rw1)
    semei = (semei0, semei1)
    semg = (semg0, semg1)
    sems = (sems0, sems1)

    def zb(i, carry):
        for k in range(D // L):
            zbuf[i, pl.ds(k * L, L)] = _z16f()
        return carry

    lax.fori_loop(0, ZROW, zb, 0)
    for c in range(RPT // ZROW):
        pltpu.sync_copy(zbuf, h_sh.at[pl.ds(sid * RPT + c * ZROW, ZROW)])
    plsc.subcore_barrier()

    def load_ei(k, p):
        base = wid * EPW + k * B
        pltpu.async_copy(ei2_hbm.at[pl.ds(base, B)], ei[p].at[0], semei[p])
        pltpu.async_copy(ei2_hbm.at[pl.ds(N_EDGES + base, B)], ei[p].at[1], semei[p])
        pltpu.async_copy(et_hbm.at[pl.ds(base, B)], ei[p].at[2], semei[p])

    def wait_ei(p):
        pltpu.make_async_copy(ei2_hbm.at[pl.ds(0, B)], ei[p].at[0],
                              semei[p]).wait()
        pltpu.make_async_copy(ei2_hbm.at[pl.ds(0, B)], ei[p].at[1],
                              semei[p]).wait()
        pltpu.make_async_copy(et_hbm.at[pl.ds(0, B)], ei[p].at[2],
                              semei[p]).wait()

    load_ei(0, 0)

    def step(k, p):
        q = 1 - p

        @pl.when(k < NCHUNK)
        def _():
            wait_ei(p)

            @pl.when(k >= 2)
            def _():
                pltpu.make_async_copy(rw[p], h_sh.at[dc[p]], sems[p]).wait()

            for j in range(B // L):
                s = pl.ds(j * L, L)
                gid[p][s] = ei[p][0, s] * N_T + ei[p][2, s]
                sg[p][s] = ei[p][1, s] * N_T + ei[p][2, s]
                dc[p][s] = ei[p][1, s]
            pltpu.async_copy(xat_hbm.at[gid[p]], rw[p], semg[p])
            pltpu.async_copy(rec_hbm.at[sg[p]], rr[p], semg[p])

        @pl.when(k >= 1)
        def _():
            pltpu.make_async_copy(xat_hbm.at[gid[q]], rw[q], semg[q]).wait()
            pltpu.make_async_copy(rec_hbm.at[sg[q]], rr[q], semg[q]).wait()

        @pl.when(k + 1 < NCHUNK)
        def _():
            load_ei(jnp.minimum(k + 1, NCHUNK - 1), q)

        @pl.when(k >= 1)
        def _():
            for g in range(B // L):
                rv = rr[q][pl.ds(g * L, L)]
                for jj in range(L):
                    j = g * L + jj
                    r16 = _bcast_lane(rv, jj)
                    for k8 in range(D // L):
                        s = pl.ds(k8 * L, L)
                        rw[q][j, s] = rw[q][j, s] * r16
            pltpu.async_copy(rw[q], h_sh.at[dc[q]], sems[q], add=True)

    def body(k2, carry):
        for b in range(2):
            step(2 * k2 + b, b)
        return carry

    lax.fori_loop(0, (NCHUNK + 1) // 2, body, 0)
    pltpu.make_async_copy(rw[0], h_sh.at[dc[0]], sems[0]).wait()
    pltpu.make_async_copy(rw[1], h_sh.at[dc[1]], sems[1]).wait()
    plsc.subcore_barrier()
    for c in range(RPT // ZROW):
        r0 = sid * RPT + c * ZROW
        pltpu.sync_copy(h_sh.at[pl.ds(r0, ZROW)], zbuf)

        @pl.when(cid == 0)
        def _():
            pltpu.sync_copy(zbuf, outp0.at[pl.ds(r0, ZROW)])

        @pl.when(cid == 1)
        def _():
            pltpu.sync_copy(zbuf, outp1.at[pl.ds(r0, ZROW)])


_scat = pl.kernel(
    _scat_body,
    out_type=(jax.ShapeDtypeStruct((N_PAD, D), jnp.float32),
              jax.ShapeDtypeStruct((N_PAD, D), jnp.float32)),
    mesh=_mesh,
    scratch_types=[
        pltpu.VMEM((3, B), jnp.int32),
        pltpu.VMEM((3, B), jnp.int32),
        pltpu.VMEM((B,), jnp.int32),
        pltpu.VMEM((B,), jnp.int32),
        pltpu.VMEM((B,), jnp.int32),
        pltpu.VMEM((B,), jnp.int32),
        pltpu.VMEM((B,), jnp.int32),
        pltpu.VMEM((B,), jnp.int32),
        pltpu.VMEM((B,), jnp.float32),
        pltpu.VMEM((B,), jnp.float32),
        pltpu.VMEM((B, D), jnp.float32),
        pltpu.VMEM((B, D), jnp.float32),
        pltpu.VMEM((ZROW, D), jnp.float32),
        pltpu.VMEM_SHARED((N_PAD, D), jnp.float32),
        pltpu.SemaphoreType.DMA,
        pltpu.SemaphoreType.DMA,
        pltpu.SemaphoreType.DMA,
        pltpu.SemaphoreType.DMA,
        pltpu.SemaphoreType.DMA,
        pltpu.SemaphoreType.DMA,
    ],
)


RB2 = 400


def _xat_body(x_ref, tw_ref, w2_ref, o_ref):
    aw = tw_ref[...] * w2_ref[...]
    o_ref[...] = (x_ref[...][:, None, :] * aw[None]).reshape(RB2 * N_T, D)


def _xat(x, tw, w2col):
    return pl.pallas_call(
        _xat_body,
        out_shape=jax.ShapeDtypeStruct((N_NODES * N_T, D), jnp.float32),
        grid=(N_NODES // RB2,),
        in_specs=[
            pl.BlockSpec((RB2, D), lambda i: (i, 0)),
            pl.BlockSpec((N_T, D), lambda i: (0, 0)),
            pl.BlockSpec((N_T, 1), lambda i: (0, 0)),
        ],
        out_specs=pl.BlockSpec((RB2 * N_T, D), lambda i: (i, 0)),
    )(x, tw, w2col)


NTR = NT_PAD // D


def _recip_body(h0_ref, h1_ref, w_ref, o_ref):
    o_ref[...] = w_ref[...] / jnp.maximum(h0_ref[...] + h1_ref[...], 1.0)


def _recip(h0, h1, wt):
    return pl.pallas_call(
        _recip_body,
        out_shape=jax.ShapeDtypeStruct((NTR, D), jnp.float32),
        grid=(1,),
        in_specs=[
            pl.BlockSpec((NTR, D), lambda i: (0, 0)),
            pl.BlockSpec((NTR, D), lambda i: (0, 0)),
            pl.BlockSpec((NTR, D), lambda i: (0, 0)),
        ],
        out_specs=pl.BlockSpec((NTR, D), lambda i: (0, 0)),
    )(h0, h1, wt)


RB = 400


def _mm_body(x_ref, h0_ref, h1_ref, w1_ref, w2_ref, b_ref, o_ref):
    hb = h0_ref[...] + h1_ref[...]
    acc = lax.dot_general(x_ref[...], w1_ref[...], (((1,), (1,)), ((), ())),
                          preferred_element_type=jnp.float32)
    acc = acc + lax.dot_general(hb, w2_ref[...], (((1,), (1,)), ((), ())),
                                preferred_element_type=jnp.float32)
    o_ref[...] = acc + b_ref[...]


def _mm(x, hp0, hp1, w1, w2, b2d):
    return pl.pallas_call(
        _mm_body,
        out_shape=jax.ShapeDtypeStruct((N_NODES, D), jnp.float32),
        grid=(N_NODES // RB,),
        in_specs=[
            pl.BlockSpec((RB, D), lambda i: (i, 0)),
            pl.BlockSpec((RB, D), lambda i: (i, 0)),
            pl.BlockSpec((RB, D), lambda i: (i, 0)),
            pl.BlockSpec((D, D), lambda i: (0, 0)),
            pl.BlockSpec((D, D), lambda i: (0, 0)),
            pl.BlockSpec((1, D), lambda i: (0, 0)),
        ],
        out_specs=pl.BlockSpec((RB, D), lambda i: (i, 0)),
    )(x, hp0, hp1, w1, w2, b2d)


def kernel(x, edge_index, edge_type, type_weights, type_weights2, W, b):
    ei2 = edge_index.astype(jnp.int32).reshape(2 * N_EDGES)
    et = edge_type.astype(jnp.int32)
    tw = type_weights[:N_T]
    w2col = type_weights2.reshape(N_T, 1).astype(jnp.float32)
    xat = _xat(x, tw, w2col)
    h0, h1 = _hist(ei2, et)
    ones = jnp.ones((NTR, D), jnp.float32)
    rec = _recip(h0.reshape(NTR, D), h1.reshape(NTR, D), ones).reshape(NT_PAD)
    hp0, hp1 = _scat(ei2, et, xat, rec)
    return _mm(x, hp0, hp1, W[:, :D], W[:, D:], b.reshape(1, D))


# speedup vs baseline: 1.1643x; 1.1643x over previous
import jax
import jax.numpy as jnp
from jax import lax
from jax.experimental import pallas as pl
from jax.experimental.pallas import tpu as pltpu
from jax.experimental.pallas import tpu_sc as plsc

N_NODES = 10000
N_EDGES = 320000
D = 128
N_T = 17
NT = N_NODES * N_T
NC, NS, L = 2, 16, 16
NW = NC * NS
EPW = N_EDGES // NW
B = 80
NCHUNK = EPW // B
ZH = 10752
NT_PAD = ZH * NS
N_PAD = 10240
RPT = N_PAD // NS
ZROW = 128

_mesh = plsc.VectorSubcoreMesh(core_axis_name="c", subcore_axis_name="s")


def _z16f():
    return jnp.zeros((L,), jnp.float32)


def _bcast_lane(vec, lane):
    idx = jnp.full((L, 1), lane, jnp.int32)
    return lax.gather(
        vec, idx,
        lax.GatherDimensionNumbers(offset_dims=(), collapsed_slice_dims=(0,),
                                   start_index_map=(0,)),
        (1,), mode=lax.GatherScatterMode.PROMISE_IN_BOUNDS)


def _hist_body(ei2_hbm, et_hbm, out0, out1,
               ei_v0, ei_v1, seg_v0, seg_v1, ones_v, zbuf, hist_sh,
               semei0, semei1, sems0, sems1):
    cid = lax.axis_index("c")
    sid = lax.axis_index("s")
    wid = cid * NS + sid
    ei = (ei_v0, ei_v1)
    seg = (seg_v0, seg_v1)
    semei = (semei0, semei1)
    sems = (sems0, sems1)

    def zb(i, carry):
        zbuf[pl.ds(i * L, L)] = _z16f()
        return carry

    lax.fori_loop(0, ZH // L, zb, 0)
    pltpu.sync_copy(zbuf, hist_sh.at[pl.ds(sid * ZH, ZH)])
    for j in range(B // L):
        ones_v[pl.ds(j * L, L)] = jnp.ones((L,), jnp.float32)
    plsc.subcore_barrier()

    def load_ei(k, p):
        base = wid * EPW + k * B
        pltpu.async_copy(ei2_hbm.at[pl.ds(N_EDGES + base, B)],
                         ei[p].at[pl.ds(0, B)], semei[p])
        pltpu.async_copy(et_hbm.at[pl.ds(base, B)],
                         ei[p].at[pl.ds(B, B)], semei[p])

    def wait_ei(p):
        pltpu.make_async_copy(et_hbm.at[pl.ds(0, 2 * B)], ei[p],
                              semei[p]).wait()

    load_ei(0, 0)

    def step(k, p):
        @pl.when(k < NCHUNK)
        def _():
            wait_ei(p)

            @pl.when(k >= 2)
            def _():
                pltpu.make_async_copy(ones_v, hist_sh.at[seg[p]],
                                      sems[p]).wait()

            for j in range(B // L):
                s = pl.ds(j * L, L)
                seg[p][s] = ei[p][pl.ds(j * L, L)] * N_T + \
                    ei[p][pl.ds(B + j * L, L)]

            @pl.when(k + 1 < NCHUNK)
            def _():
                load_ei(jnp.minimum(k + 1, NCHUNK - 1), 1 - p)

            pltpu.async_copy(ones_v, hist_sh.at[seg[p]], sems[p], add=True)

    def body(k2, carry):
        for b in range(2):
            step(2 * k2 + b, b)
        return carry

    lax.fori_loop(0, (NCHUNK + 1) // 2, body, 0)
    pltpu.make_async_copy(ones_v, hist_sh.at[seg[0]], sems[0]).wait()
    pltpu.make_async_copy(ones_v, hist_sh.at[seg[1]], sems[1]).wait()
    plsc.subcore_barrier()
    pltpu.sync_copy(hist_sh.at[pl.ds(sid * ZH, ZH)], zbuf)

    @pl.when(cid == 0)
    def _():
        pltpu.sync_copy(zbuf, out0.at[pl.ds(sid * ZH, ZH)])

    @pl.when(cid == 1)
    def _():
        pltpu.sync_copy(zbuf, out1.at[pl.ds(sid * ZH, ZH)])


_hist = pl.kernel(
    _hist_body,
    out_type=(jax.ShapeDtypeStruct((NT_PAD,), jnp.float32),
              jax.ShapeDtypeStruct((NT_PAD,), jnp.float32)),
    mesh=_mesh,
    scratch_types=[
        pltpu.VMEM((2 * B,), jnp.int32),
        pltpu.VMEM((2 * B,), jnp.int32),
        pltpu.VMEM((B,), jnp.int32),
        pltpu.VMEM((B,), jnp.int32),
        pltpu.VMEM((B,), jnp.float32),
        pltpu.VMEM((ZH,), jnp.float32),
        pltpu.VMEM_SHARED((NT_PAD,), jnp.float32),
        pltpu.SemaphoreType.DMA,
        pltpu.SemaphoreType.DMA,
        pltpu.SemaphoreType.DMA,
        pltpu.SemaphoreType.DMA,
    ],
)


def _scat_body(ei2_hbm, et_hbm, xat_hbm, rec_hbm, outp0, outp1,
               ei_v0, ei_v1, gid0, gid1, sg0, sg1, dc0, dc1, r_0, r_1,
               rw0, rw1, zbuf, h_sh,
               semei0, semei1, semg0, semg1, sems0, sems1):
    cid = lax.axis_index("c")
    sid = lax.axis_index("s")
    wid = cid * NS + sid
    ei = (ei_v0, ei_v1)
    gid = (gid0, gid1)
    sg = (sg0, sg1)
    dc = (dc0, dc1)
    rr = (r_0, r_1)
    rw = (rw0, rw1)
    semei = (semei0, semei1)
    semg = (semg0, semg1)
    sems = (sems0, sems1)

    def zb(i, carry):
        for k in range(D // L):
            zbuf[i, pl.ds(k * L, L)] = _z16f()
        return carry

    lax.fori_loop(0, ZROW, zb, 0)
    for c in range(RPT // ZROW):
        pltpu.sync_copy(zbuf, h_sh.at[pl.ds(sid * RPT + c * ZROW, ZROW)])
    plsc.subcore_barrier()

    def load_ei(k, p):
        base = wid * EPW + k * B
        pltpu.async_copy(ei2_hbm.at[pl.ds(base, B)],
                         ei[p].at[pl.ds(0, B)], semei[p])
        pltpu.async_copy(ei2_hbm.at[pl.ds(N_EDGES + base, B)],
                         ei[p].at[pl.ds(B, B)], semei[p])
        pltpu.async_copy(et_hbm.at[pl.ds(base, B)],
                         ei[p].at[pl.ds(2 * B, B)], semei[p])

    def wait_ei(p):
        pltpu.make_async_copy(ei2_hbm.at[pl.ds(0, 3 * B)], ei[p],
                              semei[p]).wait()

    load_ei(0, 0)

    def step(k, p):
        q = 1 - p

        @pl.when(k < NCHUNK)
        def _():
            wait_ei(p)

            @pl.when(k >= 2)
            def _():
                pltpu.make_async_copy(rw[p], h_sh.at[dc[p]], sems[p]).wait()

            for j in range(B // L):
                s = pl.ds(j * L, L)
                etv = ei[p][pl.ds(2 * B + j * L, L)]
                dstv = ei[p][pl.ds(B + j * L, L)]
                gid[p][s] = ei[p][pl.ds(j * L, L)] * N_T + etv
                sg[p][s] = dstv * N_T + etv
                dc[p][s] = dstv
            pltpu.async_copy(xat_hbm.at[gid[p]], rw[p], semg[p])
            pltpu.async_copy(rec_hbm.at[sg[p]], rr[p], semg[p])

        @pl.when(k >= 1)
        def _():
            pltpu.make_async_copy(xat_hbm.at[gid[q]], rw[q], semg[q]).wait()
            pltpu.make_async_copy(rec_hbm.at[sg[q]], rr[q], semg[q]).wait()

        @pl.when(k + 1 < NCHUNK)
        def _():
            load_ei(jnp.minimum(k + 1, NCHUNK - 1), q)

        @pl.when(k >= 1)
        def _():
            for g in range(B // L):
                rv = rr[q][pl.ds(g * L, L)]
                for jj in range(L):
                    j = g * L + jj
                    r16 = _bcast_lane(rv, jj)
                    for k8 in range(D // L):
                        s = pl.ds(k8 * L, L)
                        rw[q][j, s] = rw[q][j, s] * r16
            pltpu.async_copy(rw[q], h_sh.at[dc[q]], sems[q], add=True)

    def body(k2, carry):
        for b in range(2):
            step(2 * k2 + b, b)
        return carry

    lax.fori_loop(0, (NCHUNK + 1) // 2, body, 0)
    pltpu.make_async_copy(rw[0], h_sh.at[dc[0]], sems[0]).wait()
    pltpu.make_async_copy(rw[1], h_sh.at[dc[1]], sems[1]).wait()
    plsc.subcore_barrier()
    for c in range(RPT // ZROW):
        r0 = sid * RPT + c * ZROW
        pltpu.sync_copy(h_sh.at[pl.ds(r0, ZROW)], zbuf)

        @pl.when(cid == 0)
        def _():
            pltpu.sync_copy(zbuf, outp0.at[pl.ds(r0, ZROW)])

        @pl.when(cid == 1)
        def _():
            pltpu.sync_copy(zbuf, outp1.at[pl.ds(r0, ZROW)])


_scat = pl.kernel(
    _scat_body,
    out_type=(jax.ShapeDtypeStruct((N_PAD, D), jnp.float32),
              jax.ShapeDtypeStruct((N_PAD, D), jnp.float32)),
    mesh=_mesh,
    scratch_types=[
        pltpu.VMEM((3 * B,), jnp.int32),
        pltpu.VMEM((3 * B,), jnp.int32),
        pltpu.VMEM((B,), jnp.int32),
        pltpu.VMEM((B,), jnp.int32),
        pltpu.VMEM((B,), jnp.int32),
        pltpu.VMEM((B,), jnp.int32),
        pltpu.VMEM((B,), jnp.int32),
        pltpu.VMEM((B,), jnp.int32),
        pltpu.VMEM((B,), jnp.float32),
        pltpu.VMEM((B,), jnp.float32),
        pltpu.VMEM((B, D), jnp.float32),
        pltpu.VMEM((B, D), jnp.float32),
        pltpu.VMEM((ZROW, D), jnp.float32),
        pltpu.VMEM_SHARED((N_PAD, D), jnp.float32),
        pltpu.SemaphoreType.DMA,
        pltpu.SemaphoreType.DMA,
        pltpu.SemaphoreType.DMA,
        pltpu.SemaphoreType.DMA,
        pltpu.SemaphoreType.DMA,
        pltpu.SemaphoreType.DMA,
    ],
)


RB2 = 400


def _xat_body(x_ref, tw_ref, w2_ref, o_ref):
    aw = tw_ref[...] * w2_ref[...]
    o_ref[...] = (x_ref[...][:, None, :] * aw[None]).reshape(RB2 * N_T, D)


def _xat(x, tw, w2col):
    return pl.pallas_call(
        _xat_body,
        out_shape=jax.ShapeDtypeStruct((N_NODES * N_T, D), jnp.float32),
        grid=(N_NODES // RB2,),
        in_specs=[
            pl.BlockSpec((RB2, D), lambda i: (i, 0)),
            pl.BlockSpec((N_T, D), lambda i: (0, 0)),
            pl.BlockSpec((N_T, 1), lambda i: (0, 0)),
        ],
        out_specs=pl.BlockSpec((RB2 * N_T, D), lambda i: (i, 0)),
    )(x, tw, w2col)


NTR = NT_PAD // D


def _recip_body(h0_ref, h1_ref, w_ref, o_ref):
    o_ref[...] = w_ref[...] / jnp.maximum(h0_ref[...] + h1_ref[...], 1.0)


def _recip(h0, h1, wt):
    return pl.pallas_call(
        _recip_body,
        out_shape=jax.ShapeDtypeStruct((NTR, D), jnp.float32),
        grid=(1,),
        in_specs=[
            pl.BlockSpec((NTR, D), lambda i: (0, 0)),
            pl.BlockSpec((NTR, D), lambda i: (0, 0)),
            pl.BlockSpec((NTR, D), lambda i: (0, 0)),
        ],
        out_specs=pl.BlockSpec((NTR, D), lambda i: (0, 0)),
    )(h0, h1, wt)


RB = 400


def _mm_body(x_ref, h0_ref, h1_ref, w1_ref, w2_ref, b_ref, o_ref):
    hb = h0_ref[...] + h1_ref[...]
    acc = lax.dot_general(x_ref[...], w1_ref[...], (((1,), (1,)), ((), ())),
                          preferred_element_type=jnp.float32)
    acc = acc + lax.dot_general(hb, w2_ref[...], (((1,), (1,)), ((), ())),
                                preferred_element_type=jnp.float32)
    o_ref[...] = acc + b_ref[...]


def _mm(x, hp0, hp1, w1, w2, b2d):
    return pl.pallas_call(
        _mm_body,
        out_shape=jax.ShapeDtypeStruct((N_NODES, D), jnp.float32),
        grid=(N_NODES // RB,),
        in_specs=[
            pl.BlockSpec((RB, D), lambda i: (i, 0)),
            pl.BlockSpec((RB, D), lambda i: (i, 0)),
            pl.BlockSpec((RB, D), lambda i: (i, 0)),
            pl.BlockSpec((D, D), lambda i: (0, 0)),
            pl.BlockSpec((D, D), lambda i: (0, 0)),
            pl.BlockSpec((1, D), lambda i: (0, 0)),
        ],
        out_specs=pl.BlockSpec((RB, D), lambda i: (i, 0)),
    )(x, hp0, hp1, w1, w2, b2d)


def kernel(x, edge_index, edge_type, type_weights, type_weights2, W, b):
    ei2 = edge_index.astype(jnp.int32).reshape(2 * N_EDGES)
    et = edge_type.astype(jnp.int32)
    tw = type_weights[:N_T]
    w2col = type_weights2.reshape(N_T, 1).astype(jnp.float32)
    xat = _xat(x, tw, w2col)
    h0, h1 = _hist(ei2, et)
    ones = jnp.ones((NTR, D), jnp.float32)
    rec = _recip(h0.reshape(NTR, D), h1.reshape(NTR, D), ones).reshape(NT_PAD)
    hp0, hp1 = _scat(ei2, et, xat, rec)
    return _mm(x, hp0, hp1, W[:, :D], W[:, D:], b.reshape(1, D))
